# Initial kernel scaffold; baseline (speedup 1.0000x reference)
import jax
import jax.numpy as jnp
from jax.experimental import pallas as pl


def kernel(x, scale, zero):
    raise NotImplementedError("write your pallas kernel here")


# speedup vs baseline: 3.7936x; 3.7936x over previous
import functools

import jax
import jax.numpy as jnp
from jax import lax
from jax.experimental import pallas as pl
from jax.experimental.pallas import tpu as pltpu
from jax.experimental.pallas import tpu_sc as plsc

_LANES = 16
_NC = 2
_NS = 16
_NW = _NC * _NS

_N = 4096 * 4096
_N_PER_W = _N // _NW
_CHUNK = 16384
_NCH = _N_PER_W // _CHUNK

_THR = (1.0, 2.5, 3.5, 4.5, 5.5, 7.0)
_VAL = (0.0, 2.0, 3.0, 4.0, 5.0, 6.0, 8.0)


def _body(x_hbm, s_hbm, z_hbm, out_hbm, s_v, z_v, in_v, out_v):
    wid = lax.axis_index("s") * _NC + lax.axis_index("c")
    pltpu.sync_copy(s_hbm, s_v)
    pltpu.sync_copy(z_hbm, z_v)
    sv = s_v[...]
    zv = z_v[...]
    tx = [(jnp.float32(t) - zv) * sv for t in _THR]
    vx = [(jnp.float32(v) - zv) * sv for v in _VAL]
    base0 = wid * _N_PER_W

    @pl.loop(0, _NCH)
    def _chunks(g):
        base = base0 + g * _CHUNK
        pltpu.sync_copy(x_hbm.at[pl.ds(base, _CHUNK)], in_v)

        @plsc.parallel_loop(0, _CHUNK // _LANES, unroll=8)
        def _elems(i):
            xv = in_v[pl.ds(i * _LANES, _LANES)]
            r = vx[6]
            r = jnp.where(xv <= tx[5], vx[5], r)
            r = jnp.where(xv <= tx[4], vx[4], r)
            r = jnp.where(xv <= tx[3], vx[3], r)
            r = jnp.where(xv <= tx[2], vx[2], r)
            r = jnp.where(xv <= tx[1], vx[1], r)
            r = jnp.where(xv <= tx[0], vx[0], r)
            out_v[pl.ds(i * _LANES, _LANES)] = r

        pltpu.sync_copy(out_v, out_hbm.at[pl.ds(base, _CHUNK)])


_quantize = pl.kernel(
    _body,
    out_type=jax.ShapeDtypeStruct((_N,), jnp.float32),
    mesh=plsc.VectorSubcoreMesh(
        core_axis_name="c", subcore_axis_name="s",
        num_cores=_NC, num_subcores=_NS,
    ),
    scratch_types=[
        pltpu.VMEM((_LANES,), jnp.float32),
        pltpu.VMEM((_LANES,), jnp.float32),
        pltpu.VMEM((_CHUNK,), jnp.float32),
        pltpu.VMEM((_CHUNK,), jnp.float32),
    ],
)


@jax.jit
def kernel(x, scale, zero):
    s16 = jnp.broadcast_to(scale.astype(jnp.float32), (_LANES,))
    z16 = jnp.broadcast_to(zero.astype(jnp.float32), (_LANES,))
    out = _quantize(x.reshape(-1), s16, z16)
    return out.reshape(x.shape)


# speedup vs baseline: 4.8845x; 1.2876x over previous
import jax
import jax.numpy as jnp
from jax import lax
from jax.experimental import pallas as pl
from jax.experimental.pallas import tpu as pltpu
from jax.experimental.pallas import tpu_sc as plsc

_LANES = 16
_NC = 2
_NS = 16
_NW = _NC * _NS

_N = 4096 * 4096
_N_PER_W = _N // _NW
_CHUNK = 16384
_NCH = _N_PER_W // _CHUNK

_THR = (1.0, 2.5, 3.5, 4.5, 5.5, 7.0)
_VAL = (0.0, 2.0, 3.0, 4.0, 5.0, 6.0, 8.0)


def _body(x_hbm, s_hbm, z_hbm, out_hbm,
          s_v, z_v, in0, in1, ou0, ou1, is0, is1, os0, os1):
    wid = lax.axis_index("s") * _NC + lax.axis_index("c")
    pltpu.sync_copy(s_hbm, s_v)
    pltpu.sync_copy(z_hbm, z_v)
    sv = s_v[...]
    zv = z_v[...]
    tx = [(jnp.float32(t) - zv) * sv for t in _THR]
    vx = [(jnp.float32(v) - zv) * sv for v in _VAL]
    base0 = wid * _N_PER_W

    inb, oub = (in0, in1), (ou0, ou1)
    ise, ose = (is0, is1), (os0, os1)

    def start_in(g, b):
        pltpu.async_copy(x_hbm.at[pl.ds(base0 + g * _CHUNK, _CHUNK)],
                         inb[b], ise[b])

    def wait_in(b):
        pltpu.make_async_copy(x_hbm.at[pl.ds(base0, _CHUNK)],
                              inb[b], ise[b]).wait()

    def start_out(g, b):
        pltpu.async_copy(oub[b],
                         out_hbm.at[pl.ds(base0 + g * _CHUNK, _CHUNK)],
                         ose[b])

    def wait_out(b):
        pltpu.make_async_copy(oub[b],
                              out_hbm.at[pl.ds(base0, _CHUNK)],
                              ose[b]).wait()

    start_in(0, 0)
    start_in(1, 1)

    @pl.loop(0, _NCH, step=2)
    def _pair(g0):
        for b in (0, 1):
            g = g0 + b
            wait_in(b)

            @pl.when(g0 > 0)
            def _():
                wait_out(b)

            src, dst = inb[b], oub[b]

            @plsc.parallel_loop(0, _CHUNK // _LANES, unroll=8)
            def _elems(i):
                xv = src[pl.ds(i * _LANES, _LANES)]
                r = vx[6]
                r = jnp.where(xv <= tx[5], vx[5], r)
                r = jnp.where(xv <= tx[4], vx[4], r)
                r = jnp.where(xv <= tx[3], vx[3], r)
                r = jnp.where(xv <= tx[2], vx[2], r)
                r = jnp.where(xv <= tx[1], vx[1], r)
                r = jnp.where(xv <= tx[0], vx[0], r)
                dst[pl.ds(i * _LANES, _LANES)] = r

            start_out(g, b)

            @pl.when(g + 2 < _NCH)
            def _():
                start_in(g + 2, b)

    wait_out(0)
    wait_out(1)


_quantize = pl.kernel(
    _body,
    out_type=jax.ShapeDtypeStruct((_N,), jnp.float32),
    mesh=plsc.VectorSubcoreMesh(
        core_axis_name="c", subcore_axis_name="s",
        num_cores=_NC, num_subcores=_NS,
    ),
    scratch_types=[
        pltpu.VMEM((_LANES,), jnp.float32),
        pltpu.VMEM((_LANES,), jnp.float32),
        pltpu.VMEM((_CHUNK,), jnp.float32),
        pltpu.VMEM((_CHUNK,), jnp.float32),
        pltpu.VMEM((_CHUNK,), jnp.float32),
        pltpu.VMEM((_CHUNK,), jnp.float32),
        pltpu.SemaphoreType.DMA,
        pltpu.SemaphoreType.DMA,
        pltpu.SemaphoreType.DMA,
        pltpu.SemaphoreType.DMA,
    ],
)


@jax.jit
def kernel(x, scale, zero):
    s16 = jnp.broadcast_to(scale.astype(jnp.float32), (_LANES,))
    z16 = jnp.broadcast_to(zero.astype(jnp.float32), (_LANES,))
    out = _quantize(x.reshape(-1), s16, z16)
    return out.reshape(x.shape)


# speedup vs baseline: 10.5252x; 2.1548x over previous
import jax
import jax.numpy as jnp
from jax import lax
from jax.experimental import pallas as pl
from jax.experimental.pallas import tpu as pltpu
from jax.experimental.pallas import tpu_sc as plsc

_LANES = 16
_NC = 2
_NS = 16
_NW = _NC * _NS

_ROWS, _COLS = 4096, 4096
_ROWS_PER_W = _ROWS // _NW
_CROWS = 4
_NCH = _ROWS_PER_W // _CROWS

_THR = (1.0, 2.5, 3.5, 4.5, 5.5, 7.0)
_VAL = (0.0, 2.0, 3.0, 4.0, 5.0, 6.0, 8.0)


def _body(x_hbm, s_hbm, z_hbm, out_hbm,
          s_v, z_v, in0, in1, ou0, ou1, is0, is1, os0, os1):
    wid = lax.axis_index("s") * _NC + lax.axis_index("c")
    pltpu.sync_copy(s_hbm, s_v)
    pltpu.sync_copy(z_hbm, z_v)
    sv = s_v[...]
    zv = z_v[...]
    tx = [(jnp.float32(t) - zv) * sv for t in _THR]
    vx = [(jnp.float32(v) - zv) * sv for v in _VAL]
    rbase = wid * _ROWS_PER_W

    inb, oub = (in0, in1), (ou0, ou1)
    ise, ose = (is0, is1), (os0, os1)

    def start_in(g, b):
        pltpu.async_copy(x_hbm.at[pl.ds(rbase + g * _CROWS, _CROWS)],
                         inb[b], ise[b])

    def wait_in(b):
        pltpu.make_async_copy(x_hbm.at[pl.ds(rbase, _CROWS)],
                              inb[b], ise[b]).wait()

    def start_out(g, b):
        pltpu.async_copy(oub[b],
                         out_hbm.at[pl.ds(rbase + g * _CROWS, _CROWS)],
                         ose[b])

    def wait_out(b):
        pltpu.make_async_copy(oub[b],
                              out_hbm.at[pl.ds(rbase, _CROWS)],
                              ose[b]).wait()

    start_in(0, 0)
    start_in(1, 1)

    @pl.loop(0, _NCH, step=2)
    def _pair(g0):
        for b in (0, 1):
            g = g0 + b
            wait_in(b)

            @pl.when(g0 > 0)
            def _():
                wait_out(b)

            src, dst = inb[b], oub[b]

            for row in range(_CROWS):
                @plsc.parallel_loop(0, _COLS // _LANES, unroll=8)
                def _elems(i, row=row):
                    xv = src[row, pl.ds(i * _LANES, _LANES)]
                    r = vx[6]
                    r = jnp.where(xv <= tx[5], vx[5], r)
                    r = jnp.where(xv <= tx[4], vx[4], r)
                    r = jnp.where(xv <= tx[3], vx[3], r)
                    r = jnp.where(xv <= tx[2], vx[2], r)
                    r = jnp.where(xv <= tx[1], vx[1], r)
                    r = jnp.where(xv <= tx[0], vx[0], r)
                    dst[row, pl.ds(i * _LANES, _LANES)] = r

            start_out(g, b)

            @pl.when(g + 2 < _NCH)
            def _():
                start_in(g + 2, b)

    wait_out(0)
    wait_out(1)


_quantize = pl.kernel(
    _body,
    out_type=jax.ShapeDtypeStruct((_ROWS, _COLS), jnp.float32),
    mesh=plsc.VectorSubcoreMesh(
        core_axis_name="c", subcore_axis_name="s",
        num_cores=_NC, num_subcores=_NS,
    ),
    scratch_types=[
        pltpu.VMEM((_LANES,), jnp.float32),
        pltpu.VMEM((_LANES,), jnp.float32),
        pltpu.VMEM((_CROWS, _COLS), jnp.float32),
        pltpu.VMEM((_CROWS, _COLS), jnp.float32),
        pltpu.VMEM((_CROWS, _COLS), jnp.float32),
        pltpu.VMEM((_CROWS, _COLS), jnp.float32),
        pltpu.SemaphoreType.DMA,
        pltpu.SemaphoreType.DMA,
        pltpu.SemaphoreType.DMA,
        pltpu.SemaphoreType.DMA,
    ],
)


@jax.jit
def kernel(x, scale, zero):
    s16 = jnp.broadcast_to(scale.astype(jnp.float32), (_LANES,))
    z16 = jnp.broadcast_to(zero.astype(jnp.float32), (_LANES,))
    return _quantize(x, s16, z16)
